# baseline (speedup 1.0000x reference)
import functools

import jax
import jax.numpy as jnp
from jax import lax
from jax.experimental import pallas as pl
from jax.experimental.pallas import tpu as pltpu
from jax.experimental.pallas import tpu_sc as plsc

LAMB = 10.0
N_ITERS = 20
EPS = 1e-12
B, T, D = 16, 512, 512
N = T * D


def _sinkhorn_body(t_ref, d_ref, aff_ref, p_ref, rm_ref):
    b = pl.program_id(0)
    t = t_ref[b]
    d = d_ref[b]
    tf = t.astype(jnp.float32)
    df = d.astype(jnp.float32)

    aff = aff_ref[0]
    row_i = lax.broadcasted_iota(jnp.int32, (T, D), 0)
    col_j = lax.broadcasted_iota(jnp.int32, (T, D), 1)
    real = (row_i < t) & (col_j < d)
    active = (row_i <= t) & (col_j <= d)
    K = jnp.where(active, jnp.where(real, jnp.exp(LAMB * aff), 1.0), 0.0)
    KT = K.T

    rows = lax.broadcasted_iota(jnp.int32, (T, 1), 0)
    rs = jnp.where(rows < t, 1.0, jnp.where(rows == t, tf, 0.0))
    cs = jnp.where(rows < d, 1.0, jnp.where(rows == d, df, 0.0))

    v0 = jnp.ones((T, 1), jnp.float32)
    u0 = jnp.zeros((T, 1), jnp.float32)

    def body(_, carry):
        u, v = carry
        Kv = lax.dot_general(K, v, (((1,), (0,)), ((), ())),
                             preferred_element_type=jnp.float32)
        u = rs / (Kv + EPS)
        KTu = lax.dot_general(KT, u, (((1,), (0,)), ((), ())),
                              preferred_element_type=jnp.float32)
        v = cs / (KTu + EPS)
        return u, v

    u, v = lax.fori_loop(0, N_ITERS, body, (u0, v0))

    vT = v.reshape(1, T)
    P = u * (K * vT)
    rm = jnp.max(P, axis=1, keepdims=True)
    p_ref[0] = jnp.where(real, P, 0.0)
    rm_ref[0] = rm.reshape(1, T)


def _sinkhorn_dense(aff, t_arr, d_arr):
    return pl.pallas_call(
        _sinkhorn_body,
        grid=(B,),
        in_specs=[
            pl.BlockSpec(memory_space=pltpu.SMEM),
            pl.BlockSpec(memory_space=pltpu.SMEM),
            pl.BlockSpec((1, T, D), lambda b: (b, 0, 0)),
        ],
        out_specs=[
            pl.BlockSpec((1, T, D), lambda b: (b, 0, 0)),
            pl.BlockSpec((1, 1, T), lambda b: (b, 0, 0)),
        ],
        out_shape=[
            jax.ShapeDtypeStruct((B, T, D), jnp.float32),
            jax.ShapeDtypeStruct((B, 1, T), jnp.float32),
        ],
    )(t_arr, d_arr, aff)


TILE = 4096
NT_TILES = N // TILE
UNITS = B * NT_TILES
NW = 32
UPW = UNITS // NW


def _sc_pack(p_hbm, rm_hbm, t_hbm, d_hbm, rec_hbm, s_out, a_out,
             idx_p, idx_r, p_tile, rm_tile, zbuf, td_vec, rec_vec, cnt,
             psem, rsem):
    c_ax = lax.axis_index("c")
    s_ax = lax.axis_index("s")
    w = c_ax * 16 + s_ax
    lane = lax.iota(jnp.int32, 16)

    pltpu.sync_copy(t_hbm, td_vec.at[0, pl.ds(0, 16)])
    pltpu.sync_copy(d_hbm, td_vec.at[1, pl.ds(0, 16)])
    pltpu.sync_copy(rec_hbm, rec_vec.at[pl.ds(0, 16)])

    zvec = jnp.zeros((16,), jnp.float32)

    def zb_body(row, _):
        for kk in range(8):
            zbuf[row, pl.ds(kk * 16, 16)] = zvec
        return 0

    lax.fori_loop(0, 32, zb_body, 0)

    @pl.when(s_ax == 0)
    def _():
        cnt[0] = 0

    plsc.subcore_barrier()

    def unit_body(uu):
        unit = uu * 2 + c_ax
        b = unit >> 6
        tile = unit - b * NT_TILES
        k0 = tile * TILE
        t = td_vec[0, pl.ds(b, 16)][0]
        d = td_vec[1, pl.ds(b, 16)][0]
        bP = b * N
        bR = b * T
        rec = rec_vec[pl.ds(b, 16)][0]
        nvalid = jnp.clip(t * d - k0, 0, TILE)
        n128 = (nvalid + 127) >> 7

        @pl.when(n128 == 0)
        def _():
            pltpu.sync_copy(zbuf, s_out.at[unit])
            pltpu.sync_copy(zbuf, a_out.at[unit])

        @pl.when(n128 > 0)
        def _():
            def row_body(row, _):
                kbase = k0 + row * 128
                for kk in range(8):
                    kc = kbase + kk * 16 + lane
                    r0 = (kc.astype(jnp.float32) * rec).astype(jnp.int32)
                    r0 = r0 - jnp.where(r0 * d > kc, 1, 0)
                    r0 = r0 + jnp.where((r0 + 1) * d <= kc, 1, 0)
                    r = jnp.minimum(r0, t)
                    cc = jnp.where(r < t, kc - r * d, 0)
                    idx_p[row, pl.ds(kk * 16, 16)] = bP + r * D + cc
                    idx_r[row, pl.ds(kk * 16, 16)] = bR + r
                return 0

            lax.fori_loop(0, n128, row_body, 0)

            def ztail_body(row, _):
                for kk in range(8):
                    p_tile[row, pl.ds(kk * 16, 16)] = zvec
                    rm_tile[row, pl.ds(kk * 16, 16)] = zvec
                return 0

            lax.fori_loop(n128, 32, ztail_body, 0)

            def fire_body(row, _):
                pltpu.async_copy(p_hbm.at[idx_p.at[row]], p_tile.at[row], psem)
                pltpu.async_copy(rm_hbm.at[idx_r.at[row]], rm_tile.at[row],
                                 rsem)
                return 0

            lax.fori_loop(0, n128, fire_body, 0)

            def drain_body(row, _):
                pltpu.make_async_copy(p_hbm.at[idx_p.at[row]],
                                      p_tile.at[row], psem).wait()
                pltpu.make_async_copy(rm_hbm.at[idx_r.at[row]],
                                      rm_tile.at[row], rsem).wait()
                return 0

            lax.fori_loop(0, n128, drain_body, 0)

            def a_body(row, _):
                for kk in range(8):
                    p = p_tile[row, pl.ds(kk * 16, 16)]
                    rm = rm_tile[row, pl.ds(kk * 16, 16)]
                    rm_tile[row, pl.ds(kk * 16, 16)] = jnp.where(
                        p >= rm, 1.0, 0.0)
                return 0

            lax.fori_loop(0, n128, a_body, 0)
            pltpu.sync_copy(p_tile, s_out.at[unit])
            pltpu.sync_copy(rm_tile, a_out.at[unit])

    def steal_body(i, _):
        u = plsc.fetch_and_add(cnt.at[0], 1, subcore_id=0)

        @pl.when(u < UNITS // 2)
        def _():
            unit_body(u)

        return 0

    lax.fori_loop(0, UNITS // 2, steal_body, 0)


def _ragged_pack(p_flat, rm_flat, t_arr, d_arr):
    mesh = plsc.VectorSubcoreMesh(core_axis_name="c", subcore_axis_name="s")
    fn = functools.partial(
        pl.kernel,
        mesh=mesh,
        out_type=[
            jax.ShapeDtypeStruct((UNITS, 32, 128), jnp.float32),
            jax.ShapeDtypeStruct((UNITS, 32, 128), jnp.float32),
        ],
        scratch_types=[
            pltpu.VMEM((32, 128), jnp.int32),
            pltpu.VMEM((32, 128), jnp.int32),
            pltpu.VMEM((32, 128), jnp.float32),
            pltpu.VMEM((32, 128), jnp.float32),
            pltpu.VMEM((32, 128), jnp.float32),
            pltpu.VMEM((2, 32), jnp.int32),
            pltpu.VMEM((32,), jnp.float32),
            pltpu.SMEM((1,), jnp.int32),
            pltpu.SemaphoreType.DMA,
            pltpu.SemaphoreType.DMA,
        ],
    )(_sc_pack)
    rec_arr = 1.0 / d_arr.astype(jnp.float32)
    return fn(p_flat, rm_flat, t_arr, d_arr, rec_arr)


def kernel(affinity_scores, num_detections, num_tracklets):
    p, rm = _sinkhorn_dense(affinity_scores, num_tracklets, num_detections)
    s_tiles, a_tiles = _ragged_pack(
        p.reshape(B * T * D), rm.reshape(B * T), num_tracklets, num_detections)
    sinkhorn_dense = s_tiles.reshape(B, N)
    assignment_dense = a_tiles.reshape(B, N).astype(jnp.bool_)
    return sinkhorn_dense, assignment_dense


# speedup vs baseline: 1.2616x; 1.2616x over previous
import functools

import jax
import jax.numpy as jnp
from jax import lax
from jax.experimental import pallas as pl
from jax.experimental.pallas import tpu as pltpu
from jax.experimental.pallas import tpu_sc as plsc

LAMB = 10.0
N_ITERS = 20
EPS = 1e-12
B, T, D = 16, 512, 512
N = T * D


def _sinkhorn_body(t_ref, d_ref, aff_ref, p_ref, rm_ref):
    b = pl.program_id(0)
    t = t_ref[b]
    d = d_ref[b]
    tf = t.astype(jnp.float32)
    df = d.astype(jnp.float32)

    aff = aff_ref[0]
    row_i = lax.broadcasted_iota(jnp.int32, (T, D), 0)
    col_j = lax.broadcasted_iota(jnp.int32, (T, D), 1)
    real = (row_i < t) & (col_j < d)
    active = (row_i <= t) & (col_j <= d)
    K = jnp.where(active, jnp.where(real, jnp.exp(LAMB * aff), 1.0), 0.0)
    KT = K.T

    rows = lax.broadcasted_iota(jnp.int32, (T, 1), 0)
    rs = jnp.where(rows < t, 1.0, jnp.where(rows == t, tf, 0.0))
    cs = jnp.where(rows < d, 1.0, jnp.where(rows == d, df, 0.0))

    v0 = jnp.ones((T, 1), jnp.float32)
    u0 = jnp.zeros((T, 1), jnp.float32)

    def body(_, carry):
        u, v = carry
        Kv = lax.dot_general(K, v, (((1,), (0,)), ((), ())),
                             preferred_element_type=jnp.float32)
        u = rs / (Kv + EPS)
        KTu = lax.dot_general(KT, u, (((1,), (0,)), ((), ())),
                              preferred_element_type=jnp.float32)
        v = cs / (KTu + EPS)
        return u, v

    u, v = lax.fori_loop(0, N_ITERS, body, (u0, v0))

    vT = v.reshape(1, T)
    P = u * (K * vT)
    rm = jnp.max(P, axis=1, keepdims=True)
    p_ref[0] = jnp.where(real, P, 0.0)
    rm_ref[0] = rm.reshape(1, T)


def _sinkhorn_dense(aff, t_arr, d_arr):
    return pl.pallas_call(
        _sinkhorn_body,
        grid=(B,),
        in_specs=[
            pl.BlockSpec(memory_space=pltpu.SMEM),
            pl.BlockSpec(memory_space=pltpu.SMEM),
            pl.BlockSpec((1, T, D), lambda b: (b, 0, 0)),
        ],
        out_specs=[
            pl.BlockSpec((1, T, D), lambda b: (b, 0, 0)),
            pl.BlockSpec((1, 1, T), lambda b: (b, 0, 0)),
        ],
        out_shape=[
            jax.ShapeDtypeStruct((B, T, D), jnp.float32),
            jax.ShapeDtypeStruct((B, 1, T), jnp.float32),
        ],
    )(t_arr, d_arr, aff)


TILE = 4096
NT_TILES = N // TILE
UNITS = B * NT_TILES
NW = 32
UPW = UNITS // NW


def _sc_pack(p_hbm, rm_hbm, t_hbm, d_hbm, rec_hbm, s_out, a_out,
             idx_p, idx_r, p_tile, rm_tile, zbuf, td_vec, rec_vec,
             psem, rsem):
    c_ax = lax.axis_index("c")
    s_ax = lax.axis_index("s")
    w = c_ax * 16 + s_ax
    lane = lax.iota(jnp.int32, 16)

    pltpu.sync_copy(t_hbm, td_vec.at[0, pl.ds(0, 16)])
    pltpu.sync_copy(d_hbm, td_vec.at[1, pl.ds(0, 16)])
    pltpu.sync_copy(rec_hbm, rec_vec.at[pl.ds(0, 16)])

    zvec = jnp.zeros((16,), jnp.float32)

    def zb_body(row, _):
        for kk in range(8):
            zbuf[row, pl.ds(kk * 16, 16)] = zvec
        return 0

    lax.fori_loop(0, 32, zb_body, 0)

    def process_valid(b, tile):
        unit = b * NT_TILES + tile
        k0 = tile * TILE
        t = td_vec[0, pl.ds(b, 16)][0]
        d = td_vec[1, pl.ds(b, 16)][0]
        bP = b * N
        bR = b * T
        rec = rec_vec[pl.ds(b, 16)][0]
        nvalid = jnp.clip(t * d - k0, 0, TILE)
        n128 = (nvalid + 127) >> 7

        def row_body(row, _):
            kbase = k0 + row * 128
            for kk in range(8):
                kc = kbase + kk * 16 + lane
                r0 = (kc.astype(jnp.float32) * rec).astype(jnp.int32)
                r0 = r0 - jnp.where(r0 * d > kc, 1, 0)
                r0 = r0 + jnp.where((r0 + 1) * d <= kc, 1, 0)
                r = jnp.minimum(r0, t)
                cc = jnp.where(r < t, kc - r * d, 0)
                idx_p[row, pl.ds(kk * 16, 16)] = bP + r * D + cc
                idx_r[row, pl.ds(kk * 16, 16)] = bR + r
            return 0

        lax.fori_loop(0, n128, row_body, 0)

        def ztail_body(row, _):
            for kk in range(8):
                p_tile[row, pl.ds(kk * 16, 16)] = zvec
                rm_tile[row, pl.ds(kk * 16, 16)] = zvec
            return 0

        lax.fori_loop(n128, 32, ztail_body, 0)

        def fire_body(row, _):
            pltpu.async_copy(p_hbm.at[idx_p.at[row]], p_tile.at[row], psem)
            pltpu.async_copy(rm_hbm.at[idx_r.at[row]], rm_tile.at[row], rsem)
            return 0

        lax.fori_loop(0, n128, fire_body, 0)

        def drain_body(row, _):
            pltpu.make_async_copy(p_hbm.at[idx_p.at[row]],
                                  p_tile.at[row], psem).wait()
            pltpu.make_async_copy(rm_hbm.at[idx_r.at[row]],
                                  rm_tile.at[row], rsem).wait()
            return 0

        lax.fori_loop(0, n128, drain_body, 0)

        def a_body(row, _):
            for kk in range(8):
                p = p_tile[row, pl.ds(kk * 16, 16)]
                rm = rm_tile[row, pl.ds(kk * 16, 16)]
                rm_tile[row, pl.ds(kk * 16, 16)] = jnp.where(p >= rm, 1.0, 0.0)
            return 0

        lax.fori_loop(0, n128, a_body, 0)
        pltpu.sync_copy(p_tile, s_out.at[unit])
        pltpu.sync_copy(rm_tile, a_out.at[unit])

    def nvalid_tiles(b):
        t = td_vec[0, pl.ds(b, 16)][0]
        d = td_vec[1, pl.ds(b, 16)][0]
        return jnp.minimum((t * d + TILE - 1) >> 12, NT_TILES)

    def vb_body(b, s):
        vb = nvalid_tiles(b)
        start = (w - s) & 31
        nj = (vb - start + 31) >> 5

        def vt_body(j, _):
            process_valid(b, start + 32 * j)
            return 0

        lax.fori_loop(0, nj, vt_body, 0)
        return s + vb

    lax.fori_loop(0, B, vb_body, 0)

    def pb_body(b, carry):
        s, fired = carry
        vb = nvalid_tiles(b)
        npad = NT_TILES - vb
        start = (w - s) & 31
        nj = (npad - start + 31) >> 5

        def pt_body(j, f):
            unit = b * NT_TILES + vb + start + 32 * j
            pltpu.async_copy(zbuf, s_out.at[unit], psem)
            pltpu.async_copy(zbuf, a_out.at[unit], psem)
            return f + 2

        fired = lax.fori_loop(0, nj, pt_body, fired)
        return s + npad, fired

    _, fired = lax.fori_loop(0, B, pb_body, (0, 0))

    def pdrain_body(i, _):
        pltpu.make_async_copy(zbuf, s_out.at[0], psem).wait()
        return 0

    lax.fori_loop(0, fired, pdrain_body, 0)


def _ragged_pack(p_flat, rm_flat, t_arr, d_arr):
    mesh = plsc.VectorSubcoreMesh(core_axis_name="c", subcore_axis_name="s")
    fn = functools.partial(
        pl.kernel,
        mesh=mesh,
        out_type=[
            jax.ShapeDtypeStruct((UNITS, 32, 128), jnp.float32),
            jax.ShapeDtypeStruct((UNITS, 32, 128), jnp.float32),
        ],
        scratch_types=[
            pltpu.VMEM((32, 128), jnp.int32),
            pltpu.VMEM((32, 128), jnp.int32),
            pltpu.VMEM((32, 128), jnp.float32),
            pltpu.VMEM((32, 128), jnp.float32),
            pltpu.VMEM((32, 128), jnp.float32),
            pltpu.VMEM((2, 32), jnp.int32),
            pltpu.VMEM((32,), jnp.float32),
            pltpu.SemaphoreType.DMA,
            pltpu.SemaphoreType.DMA,
        ],
    )(_sc_pack)
    rec_arr = 1.0 / d_arr.astype(jnp.float32)
    return fn(p_flat, rm_flat, t_arr, d_arr, rec_arr)


def kernel(affinity_scores, num_detections, num_tracklets):
    p, rm = _sinkhorn_dense(affinity_scores, num_tracklets, num_detections)
    s_tiles, a_tiles = _ragged_pack(
        p.reshape(B * T * D), rm.reshape(B * T), num_tracklets, num_detections)
    sinkhorn_dense = s_tiles.reshape(B, N)
    assignment_dense = a_tiles.reshape(B, N).astype(jnp.bool_)
    return sinkhorn_dense, assignment_dense


# speedup vs baseline: 3.5957x; 2.8502x over previous
import functools

import jax
import jax.numpy as jnp
from jax import lax
from jax.experimental import pallas as pl
from jax.experimental.pallas import tpu as pltpu
from jax.experimental.pallas import tpu_sc as plsc

LAMB = 10.0
N_ITERS = 20
EPS = 1e-12
B, T, D = 16, 512, 512
N = T * D


def _sinkhorn_body(t_ref, d_ref, aff_ref, p_ref, rm_ref):
    b = pl.program_id(0)
    t = t_ref[b]
    d = d_ref[b]
    tf = t.astype(jnp.float32)
    df = d.astype(jnp.float32)

    aff = aff_ref[0]
    row_i = lax.broadcasted_iota(jnp.int32, (T, D), 0)
    col_j = lax.broadcasted_iota(jnp.int32, (T, D), 1)
    real = (row_i < t) & (col_j < d)
    active = (row_i <= t) & (col_j <= d)
    K = jnp.where(active, jnp.where(real, jnp.exp(LAMB * aff), 1.0), 0.0)
    KT = K.T

    rows = lax.broadcasted_iota(jnp.int32, (T, 1), 0)
    rs = jnp.where(rows < t, 1.0, jnp.where(rows == t, tf, 0.0))
    cs = jnp.where(rows < d, 1.0, jnp.where(rows == d, df, 0.0))

    v0 = jnp.ones((T, 1), jnp.float32)
    u0 = jnp.zeros((T, 1), jnp.float32)

    def body(_, carry):
        u, v = carry
        Kv = lax.dot_general(K, v, (((1,), (0,)), ((), ())),
                             preferred_element_type=jnp.float32)
        u = rs / (Kv + EPS)
        KTu = lax.dot_general(KT, u, (((1,), (0,)), ((), ())),
                              preferred_element_type=jnp.float32)
        v = cs / (KTu + EPS)
        return u, v

    u, v = lax.fori_loop(0, N_ITERS, body, (u0, v0))

    vT = v.reshape(1, T)
    P = u * (K * vT)
    rm = jnp.max(P, axis=1, keepdims=True)
    p_ref[0] = jnp.where(real, P, 0.0)
    rm_ref[0] = rm.reshape(1, T)


def _sinkhorn_dense(aff, t_arr, d_arr):
    return pl.pallas_call(
        _sinkhorn_body,
        grid=(B,),
        in_specs=[
            pl.BlockSpec(memory_space=pltpu.SMEM),
            pl.BlockSpec(memory_space=pltpu.SMEM),
            pl.BlockSpec((1, T, D), lambda b: (b, 0, 0)),
        ],
        out_specs=[
            pl.BlockSpec((1, T, D), lambda b: (b, 0, 0)),
            pl.BlockSpec((1, 1, T), lambda b: (b, 0, 0)),
        ],
        out_shape=[
            jax.ShapeDtypeStruct((B, T, D), jnp.float32),
            jax.ShapeDtypeStruct((B, 1, T), jnp.float32),
        ],
    )(t_arr, d_arr, aff)


TILE = 4096
NT_TILES = N // TILE
UNITS = B * NT_TILES
NW = 32
UPW = UNITS // NW


def _sc_pack(p_hbm, rm_hbm, t_hbm, d_hbm, rec_hbm, s_out, a_out,
             idx_p, idx_r, p_tile, rm_tile, a_tile, rm_row, zbuf, td_vec,
             rec_vec, psem, rsem):
    c_ax = lax.axis_index("c")
    s_ax = lax.axis_index("s")
    w = c_ax * 16 + s_ax
    lane = lax.iota(jnp.int32, 16)

    pltpu.sync_copy(t_hbm, td_vec.at[0, pl.ds(0, 16)])
    pltpu.sync_copy(d_hbm, td_vec.at[1, pl.ds(0, 16)])
    pltpu.sync_copy(rec_hbm, rec_vec.at[pl.ds(0, 16)])

    zvec = jnp.zeros((16,), jnp.float32)

    def zb_body(row, _):
        for kk in range(8):
            zbuf[row, pl.ds(kk * 16, 16)] = zvec
        return 0

    lax.fori_loop(0, 32, zb_body, 0)

    def process_valid(b, tile):
        unit = b * NT_TILES + tile
        k0 = tile * TILE
        t = td_vec[0, pl.ds(b, 16)][0]
        d = td_vec[1, pl.ds(b, 16)][0]
        bP = b * N
        bR = b * T
        rec = rec_vec[pl.ds(b, 16)][0]
        nvalid = jnp.clip(t * d - k0, 0, TILE)
        n128 = (nvalid + 127) >> 7

        def sdiv(kc0):
            r0 = (kc0.astype(jnp.float32) * rec).astype(jnp.int32)
            r0 = r0 - jnp.where(r0 * d > kc0, 1, 0)
            r0 = r0 + jnp.where((r0 + 1) * d <= kc0, 1, 0)
            return jnp.minimum(r0, t)

        def row_body(row, _):
            kbase = k0 + row * 128
            for kk in range(8):
                kc = kbase + kk * 16 + lane
                r0 = (kc.astype(jnp.float32) * rec).astype(jnp.int32)
                r0 = r0 - jnp.where(r0 * d > kc, 1, 0)
                r0 = r0 + jnp.where((r0 + 1) * d <= kc, 1, 0)
                r = jnp.minimum(r0, t)
                cc = jnp.where(r < t, kc - r * d, 0)
                idx_p[row, pl.ds(kk * 16, 16)] = bP + r * D + cc
            return 0

        lax.fori_loop(0, n128, row_body, 0)

        def ztail_body(row, _):
            for kk in range(8):
                p_tile[row, pl.ds(kk * 16, 16)] = zvec
                rm_tile[row, pl.ds(kk * 16, 16)] = zvec
            return 0

        lax.fori_loop(n128, 32, ztail_body, 0)

        def fire_body(row, _):
            pltpu.async_copy(p_hbm.at[idx_p.at[row]], p_tile.at[row], psem)
            return 0

        lax.fori_loop(0, n128, fire_body, 0)
        pltpu.sync_copy(rm_hbm.at[pl.ds(bR, T)], rm_row.at[pl.ds(0, T)])

        def drain_body(row, _):
            pltpu.make_async_copy(p_hbm.at[idx_p.at[row]],
                                  p_tile.at[row], psem).wait()
            return 0

        lax.fori_loop(0, n128, drain_body, 0)

        @pl.when(d >= 16)
        def _():
            def a_body(row, _):
                kbase = k0 + row * 128
                for kk in range(8):
                    kc0 = kbase + kk * 16
                    r_lo = sdiv(kc0)
                    r_hi = sdiv(kc0 + 15)
                    rm_lo = rm_row[pl.ds(r_lo, 16)][0]
                    rm_hi = rm_row[pl.ds(r_hi, 16)][0]
                    rmv = jnp.where(kc0 + lane >= r_hi * d, rm_hi, rm_lo)
                    p = p_tile[row, pl.ds(kk * 16, 16)]
                    rm_tile[row, pl.ds(kk * 16, 16)] = jnp.where(
                        p >= rmv, 1.0, 0.0)
                return 0

            lax.fori_loop(0, n128, a_body, 0)

        @pl.when(d < 16)
        def _():
            def ri_body(row, _):
                kbase = k0 + row * 128
                for kk in range(8):
                    kc = kbase + kk * 16 + lane
                    r0 = (kc.astype(jnp.float32) * rec).astype(jnp.int32)
                    r0 = r0 - jnp.where(r0 * d > kc, 1, 0)
                    r0 = r0 + jnp.where((r0 + 1) * d <= kc, 1, 0)
                    idx_r[row, pl.ds(kk * 16, 16)] = bR + jnp.minimum(r0, t)
                pltpu.async_copy(rm_hbm.at[idx_r.at[row]], a_tile.at[row],
                                 rsem)
                return 0

            lax.fori_loop(0, n128, ri_body, 0)

            def rdrain_body(row, _):
                pltpu.make_async_copy(rm_hbm.at[idx_r.at[row]],
                                      a_tile.at[row], rsem).wait()
                for kk in range(8):
                    p = p_tile[row, pl.ds(kk * 16, 16)]
                    rm = a_tile[row, pl.ds(kk * 16, 16)]
                    rm_tile[row, pl.ds(kk * 16, 16)] = jnp.where(
                        p >= rm, 1.0, 0.0)
                return 0

            lax.fori_loop(0, n128, rdrain_body, 0)

        pltpu.sync_copy(p_tile, s_out.at[unit])
        pltpu.sync_copy(rm_tile, a_out.at[unit])

    def nvalid_tiles(b):
        t = td_vec[0, pl.ds(b, 16)][0]
        d = td_vec[1, pl.ds(b, 16)][0]
        return jnp.minimum((t * d + TILE - 1) >> 12, NT_TILES)

    def vb_body(b, s):
        vb = nvalid_tiles(b)
        start = (w - s) & 31
        nj = (vb - start + 31) >> 5

        def vt_body(j, _):
            process_valid(b, start + 32 * j)
            return 0

        lax.fori_loop(0, nj, vt_body, 0)
        return s + vb

    lax.fori_loop(0, B, vb_body, 0)

    def pb_body(b, carry):
        s, fired = carry
        vb = nvalid_tiles(b)
        npad = NT_TILES - vb
        start = (w - s) & 31
        nj = (npad - start + 31) >> 5

        def pt_body(j, f):
            unit = b * NT_TILES + vb + start + 32 * j
            pltpu.async_copy(zbuf, s_out.at[unit], psem)
            pltpu.async_copy(zbuf, a_out.at[unit], psem)
            return f + 2

        fired = lax.fori_loop(0, nj, pt_body, fired)
        return s + npad, fired

    _, fired = lax.fori_loop(0, B, pb_body, (0, 0))

    def pdrain_body(i, _):
        pltpu.make_async_copy(zbuf, s_out.at[0], psem).wait()
        return 0

    lax.fori_loop(0, fired, pdrain_body, 0)


def _ragged_pack(p_flat, rm_flat, t_arr, d_arr):
    mesh = plsc.VectorSubcoreMesh(core_axis_name="c", subcore_axis_name="s")
    fn = functools.partial(
        pl.kernel,
        mesh=mesh,
        out_type=[
            jax.ShapeDtypeStruct((UNITS, 32, 128), jnp.float32),
            jax.ShapeDtypeStruct((UNITS, 32, 128), jnp.float32),
        ],
        scratch_types=[
            pltpu.VMEM((32, 128), jnp.int32),
            pltpu.VMEM((32, 128), jnp.int32),
            pltpu.VMEM((32, 128), jnp.float32),
            pltpu.VMEM((32, 128), jnp.float32),
            pltpu.VMEM((32, 128), jnp.float32),
            pltpu.VMEM((T + 16,), jnp.float32),
            pltpu.VMEM((32, 128), jnp.float32),
            pltpu.VMEM((2, 32), jnp.int32),
            pltpu.VMEM((32,), jnp.float32),
            pltpu.SemaphoreType.DMA,
            pltpu.SemaphoreType.DMA,
        ],
    )(_sc_pack)
    rec_arr = 1.0 / d_arr.astype(jnp.float32)
    return fn(p_flat, rm_flat, t_arr, d_arr, rec_arr)


def kernel(affinity_scores, num_detections, num_tracklets):
    p, rm = _sinkhorn_dense(affinity_scores, num_tracklets, num_detections)
    s_tiles, a_tiles = _ragged_pack(
        p.reshape(B * T * D), rm.reshape(B * T), num_tracklets, num_detections)
    sinkhorn_dense = s_tiles.reshape(B, N)
    assignment_dense = a_tiles.reshape(B, N).astype(jnp.bool_)
    return sinkhorn_dense, assignment_dense


# speedup vs baseline: 4.6693x; 1.2986x over previous
import functools

import jax
import jax.numpy as jnp
from jax import lax
from jax.experimental import pallas as pl
from jax.experimental.pallas import tpu as pltpu
from jax.experimental.pallas import tpu_sc as plsc

LAMB = 10.0
N_ITERS = 20
EPS = 1e-12
B, T, D = 16, 512, 512
N = T * D


G = 2


def _sinkhorn_body(t_ref, d_ref, aff_ref, p_ref, rm_ref):
    bg = pl.program_id(0)
    row_i = lax.broadcasted_iota(jnp.int32, (T, D), 0)
    col_j = lax.broadcasted_iota(jnp.int32, (T, D), 1)
    rows = lax.broadcasted_iota(jnp.int32, (T, 1), 0)

    Ks, KTs, rss, css, reals = [], [], [], [], []
    for g in range(G):
        t = t_ref[bg * G + g]
        d = d_ref[bg * G + g]
        real = (row_i < t) & (col_j < d)
        active = (row_i <= t) & (col_j <= d)
        K = jnp.where(active, jnp.where(real, jnp.exp(LAMB * aff_ref[g]),
                                        1.0), 0.0)
        Ks.append(K)
        KTs.append(K.T)
        rss.append(jnp.where(rows < t, 1.0,
                             jnp.where(rows == t, t.astype(jnp.float32), 0.0)))
        css.append(jnp.where(rows < d, 1.0,
                             jnp.where(rows == d, d.astype(jnp.float32), 0.0)))
        reals.append(real)

    v0 = jnp.ones((T, 1), jnp.float32)
    u0 = jnp.zeros((T, 1), jnp.float32)

    def body(_, carry):
        us, vs = carry
        nus, nvs = [], []
        for g in range(G):
            Kv = lax.dot_general(Ks[g], vs[g], (((1,), (0,)), ((), ())),
                                 preferred_element_type=jnp.float32)
            nus.append(rss[g] / (Kv + EPS))
        for g in range(G):
            KTu = lax.dot_general(KTs[g], nus[g], (((1,), (0,)), ((), ())),
                                  preferred_element_type=jnp.float32)
            nvs.append(css[g] / (KTu + EPS))
        return tuple(nus), tuple(nvs)

    us, vs = lax.fori_loop(0, N_ITERS, body, ((u0,) * G, (v0,) * G))

    for g in range(G):
        vT = vs[g].reshape(1, T)
        P = us[g] * (Ks[g] * vT)
        rm = jnp.max(P, axis=1, keepdims=True)
        p_ref[g] = jnp.where(reals[g], P, 0.0)
        rm_ref[g] = rm.reshape(1, T)


def _sinkhorn_dense(aff, t_arr, d_arr):
    return pl.pallas_call(
        _sinkhorn_body,
        grid=(B // G,),
        in_specs=[
            pl.BlockSpec(memory_space=pltpu.SMEM),
            pl.BlockSpec(memory_space=pltpu.SMEM),
            pl.BlockSpec((G, T, D), lambda b: (b, 0, 0)),
        ],
        out_specs=[
            pl.BlockSpec((G, T, D), lambda b: (b, 0, 0)),
            pl.BlockSpec((G, 1, T), lambda b: (b, 0, 0)),
        ],
        out_shape=[
            jax.ShapeDtypeStruct((B, T, D), jnp.float32),
            jax.ShapeDtypeStruct((B, 1, T), jnp.float32),
        ],
    )(t_arr, d_arr, aff)


TILE = 4096
NT_TILES = N // TILE
UNITS = B * NT_TILES
NW = 32
UPW = UNITS // NW


def _sc_pack(p_hbm, rm_hbm, t_hbm, d_hbm, rec_hbm, s_out, a_out,
             idx_p, idx_r, p_tile, rm_tile, a_tile, rm_row, zbuf, td_vec,
             rec_vec, psem, rsem):
    c_ax = lax.axis_index("c")
    s_ax = lax.axis_index("s")
    w = c_ax * 16 + s_ax
    lane = lax.iota(jnp.int32, 16)

    pltpu.sync_copy(t_hbm, td_vec.at[0, pl.ds(0, 16)])
    pltpu.sync_copy(d_hbm, td_vec.at[1, pl.ds(0, 16)])
    pltpu.sync_copy(rec_hbm, rec_vec.at[pl.ds(0, 16)])

    zvec = jnp.zeros((16,), jnp.float32)

    def zb_body(row, _):
        for kk in range(8):
            zbuf[row, pl.ds(kk * 16, 16)] = zvec
        return 0

    lax.fori_loop(0, 32, zb_body, 0)

    def process_valid(b, tile):
        unit = b * NT_TILES + tile
        k0 = tile * TILE
        t = td_vec[0, pl.ds(b, 16)][0]
        d = td_vec[1, pl.ds(b, 16)][0]
        bP = b * N
        bR = b * T
        rec = rec_vec[pl.ds(b, 16)][0]
        nvalid = jnp.clip(t * d - k0, 0, TILE)
        n128 = (nvalid + 127) >> 7

        def sdiv(kc0):
            r0 = (kc0.astype(jnp.float32) * rec).astype(jnp.int32)
            r0 = r0 - jnp.where(r0 * d > kc0, 1, 0)
            r0 = r0 + jnp.where((r0 + 1) * d <= kc0, 1, 0)
            return jnp.minimum(r0, t)

        def row_body(row, _):
            kbase = k0 + row * 128
            for kk in range(8):
                kc = kbase + kk * 16 + lane
                r0 = (kc.astype(jnp.float32) * rec).astype(jnp.int32)
                r0 = r0 - jnp.where(r0 * d > kc, 1, 0)
                r0 = r0 + jnp.where((r0 + 1) * d <= kc, 1, 0)
                r = jnp.minimum(r0, t)
                cc = jnp.where(r < t, kc - r * d, 0)
                idx_p[row, pl.ds(kk * 16, 16)] = bP + r * D + cc
            return 0

        lax.fori_loop(0, n128, row_body, 0)

        def ztail_body(row, _):
            for kk in range(8):
                p_tile[row, pl.ds(kk * 16, 16)] = zvec
                rm_tile[row, pl.ds(kk * 16, 16)] = zvec
            return 0

        lax.fori_loop(n128, 32, ztail_body, 0)

        def fire_body(row, _):
            pltpu.async_copy(p_hbm.at[idx_p.at[row]], p_tile.at[row], psem)
            return 0

        lax.fori_loop(0, n128, fire_body, 0)
        pltpu.sync_copy(rm_hbm.at[pl.ds(bR, T)], rm_row.at[pl.ds(0, T)])

        def drain_body(row, _):
            pltpu.make_async_copy(p_hbm.at[idx_p.at[row]],
                                  p_tile.at[row], psem).wait()
            return 0

        lax.fori_loop(0, n128, drain_body, 0)

        @pl.when(d >= 16)
        def _():
            def a_body(row, _):
                kbase = k0 + row * 128
                for kk in range(8):
                    kc0 = kbase + kk * 16
                    r_lo = sdiv(kc0)
                    r_hi = sdiv(kc0 + 15)
                    rm_lo = rm_row[pl.ds(r_lo, 16)][0]
                    rm_hi = rm_row[pl.ds(r_hi, 16)][0]
                    rmv = jnp.where(kc0 + lane >= r_hi * d, rm_hi, rm_lo)
                    p = p_tile[row, pl.ds(kk * 16, 16)]
                    rm_tile[row, pl.ds(kk * 16, 16)] = jnp.where(
                        p >= rmv, 1.0, 0.0)
                return 0

            lax.fori_loop(0, n128, a_body, 0)

        @pl.when(d < 16)
        def _():
            def ri_body(row, _):
                kbase = k0 + row * 128
                for kk in range(8):
                    kc = kbase + kk * 16 + lane
                    r0 = (kc.astype(jnp.float32) * rec).astype(jnp.int32)
                    r0 = r0 - jnp.where(r0 * d > kc, 1, 0)
                    r0 = r0 + jnp.where((r0 + 1) * d <= kc, 1, 0)
                    idx_r[row, pl.ds(kk * 16, 16)] = bR + jnp.minimum(r0, t)
                pltpu.async_copy(rm_hbm.at[idx_r.at[row]], a_tile.at[row],
                                 rsem)
                return 0

            lax.fori_loop(0, n128, ri_body, 0)

            def rdrain_body(row, _):
                pltpu.make_async_copy(rm_hbm.at[idx_r.at[row]],
                                      a_tile.at[row], rsem).wait()
                for kk in range(8):
                    p = p_tile[row, pl.ds(kk * 16, 16)]
                    rm = a_tile[row, pl.ds(kk * 16, 16)]
                    rm_tile[row, pl.ds(kk * 16, 16)] = jnp.where(
                        p >= rm, 1.0, 0.0)
                return 0

            lax.fori_loop(0, n128, rdrain_body, 0)

        pltpu.sync_copy(p_tile, s_out.at[unit])
        pltpu.sync_copy(rm_tile, a_out.at[unit])

    def nvalid_tiles(b):
        t = td_vec[0, pl.ds(b, 16)][0]
        d = td_vec[1, pl.ds(b, 16)][0]
        return jnp.minimum((t * d + TILE - 1) >> 12, NT_TILES)

    def vb_body(b, s):
        vb = nvalid_tiles(b)
        start = (w - s) & 31
        nj = (vb - start + 31) >> 5

        def vt_body(j, _):
            process_valid(b, start + 32 * j)
            return 0

        lax.fori_loop(0, nj, vt_body, 0)
        return s + vb

    lax.fori_loop(0, B, vb_body, 0)

    def pb_body(b, carry):
        s, fired = carry
        vb = nvalid_tiles(b)
        npad = NT_TILES - vb
        start = (w - s) & 31
        nj = (npad - start + 31) >> 5

        def pt_body(j, f):
            unit = b * NT_TILES + vb + start + 32 * j
            pltpu.async_copy(zbuf, s_out.at[unit], psem)
            pltpu.async_copy(zbuf, a_out.at[unit], psem)
            return f + 2

        fired = lax.fori_loop(0, nj, pt_body, fired)
        return s + npad, fired

    _, fired = lax.fori_loop(0, B, pb_body, (0, 0))

    def pdrain_body(i, _):
        pltpu.make_async_copy(zbuf, s_out.at[0], psem).wait()
        return 0

    lax.fori_loop(0, fired, pdrain_body, 0)


def _ragged_pack(p_flat, rm_flat, t_arr, d_arr):
    mesh = plsc.VectorSubcoreMesh(core_axis_name="c", subcore_axis_name="s")
    fn = functools.partial(
        pl.kernel,
        mesh=mesh,
        out_type=[
            jax.ShapeDtypeStruct((UNITS, 32, 128), jnp.float32),
            jax.ShapeDtypeStruct((UNITS, 32, 128), jnp.float32),
        ],
        scratch_types=[
            pltpu.VMEM((32, 128), jnp.int32),
            pltpu.VMEM((32, 128), jnp.int32),
            pltpu.VMEM((32, 128), jnp.float32),
            pltpu.VMEM((32, 128), jnp.float32),
            pltpu.VMEM((32, 128), jnp.float32),
            pltpu.VMEM((T + 16,), jnp.float32),
            pltpu.VMEM((32, 128), jnp.float32),
            pltpu.VMEM((2, 32), jnp.int32),
            pltpu.VMEM((32,), jnp.float32),
            pltpu.SemaphoreType.DMA,
            pltpu.SemaphoreType.DMA,
        ],
    )(_sc_pack)
    rec_arr = 1.0 / d_arr.astype(jnp.float32)
    return fn(p_flat, rm_flat, t_arr, d_arr, rec_arr)


def kernel(affinity_scores, num_detections, num_tracklets):
    p, rm = _sinkhorn_dense(affinity_scores, num_tracklets, num_detections)
    s_tiles, a_tiles = _ragged_pack(
        p.reshape(B * T * D), rm.reshape(B * T), num_tracklets, num_detections)
    sinkhorn_dense = s_tiles.reshape(B, N)
    assignment_dense = a_tiles.reshape(B, N).astype(jnp.bool_)
    return sinkhorn_dense, assignment_dense
